# baseline (speedup 1.0000x reference)
import functools
import math

import jax
import jax.numpy as jnp
from jax import lax
from jax.experimental import pallas as pl
from jax.experimental.pallas import tpu as pltpu
from jax.experimental.pallas import tpu_sc as plsc

K = 16
H = 4
RADIUS = 50.0

BM_A = 256
BM_C = 256
CH = 128


def _kernel_a(feats, crow, ccol, wfp_t, wq_t, wkt_t, wvt_t, wqp,
              bfp, bq, bkt, bvt,
              bf_o, q_o, kt_o, vt_o, qp_o, idx_o, knn_o, val_o):
    i = pl.program_id(1)
    n_total = crow.shape[2]

    x = feats[0]
    bf = jnp.dot(x, wfp_t[...], preferred_element_type=jnp.float32) + bfp[...]
    q = jnp.dot(bf, wq_t[...], preferred_element_type=jnp.float32) + bq[...]
    kt = jnp.dot(bf, wkt_t[...], preferred_element_type=jnp.float32) + bkt[...]
    vt = jnp.dot(bf, wvt_t[...], preferred_element_type=jnp.float32) + bvt[...]
    qp = jnp.dot(q, wqp[...], preferred_element_type=jnp.float32)
    bf_o[0] = bf
    q_o[0] = q
    kt_o[0] = kt
    vt_o[0] = vt
    qp_o[0] = qp

    cr = crow[0]
    cc = ccol[0]
    xr = cr[0:1, :]
    yr = cr[1:2, :]
    zr = cr[2:3, :]
    xc = cc[:, 0:1]
    yc = cc[:, 1:2]
    zc = cc[:, 2:3]
    dx = xc - xr
    dy = yc - yr
    dz = zc - zr
    sq = dx * dx + dy * dy
    pos = sq > 0.0
    spatial = jnp.where(pos, jnp.sqrt(jnp.where(pos, sq, 1.0)), 0.0)
    dist = spatial + 0.3 * jnp.abs(dz)

    jj = lax.broadcasted_iota(jnp.int32, dist.shape, 1)
    nn = lax.broadcasted_iota(jnp.int32, dist.shape, 0) + i * BM_A
    inf = jnp.float32(jnp.inf)
    masked = (jj == nn) | (zr > zc) | (spatial > RADIUS)
    d = jnp.where(masked, inf, dist)

    idx_cols = []
    knn_cols = []
    val_cols = []
    for _ in range(K):
        m = jnp.min(d, axis=1, keepdims=True)
        eq = d == m
        cand = jnp.where(eq, jj, n_total)
        sel = jnp.min(cand, axis=1, keepdims=True)
        valid = m < inf
        idx_cols.append(sel)
        knn_cols.append(jnp.where(valid, m, 0.0))
        val_cols.append(valid.astype(jnp.float32))
        d = jnp.where(jj == sel, inf, d)
    idx_o[0] = jnp.concatenate(idx_cols, axis=1)
    knn_o[0] = jnp.concatenate(knn_cols, axis=1)
    val_o[0] = jnp.concatenate(val_cols, axis=1)


def _kernel_c(q_r, qp_r, bf_r, g_r, own_r,
              knn_r, val_r,
              wpe1_t, bpe1, wv2bd, wout_t, bout, gamma, beta, out_r):
    e = q_r.shape[2]
    head = e // H
    q = q_r[0]
    qp = qp_r[0]
    bf = bf_r[0]
    g = g_r[0]
    ktg = g[:, :, 0:e]
    vtg = g[:, :, e:2 * e]
    own = own_r[0]
    knn = knn_r[0]
    val = val_r[0]

    relx = g[:, :, 2 * e:2 * e + 1] - own[:, None, 0:1]
    rely = g[:, :, 2 * e + 1:2 * e + 2] - own[:, None, 1:2]
    relz = g[:, :, 2 * e + 2:2 * e + 3] - own[:, None, 2:3]
    acc = jnp.broadcast_to(jnp.reshape(bpe1[...], (1, 1, 2 * head)),
                           (q.shape[0], K, 2 * head))
    feats4 = [relx, rely, relz, knn[:, :, None]]
    for f in range(4):
        acc = acc + feats4[f] * jnp.reshape(wpe1_t[f:f + 1, :], (1, 1, 2 * head))
    hrel = jnp.maximum(acc, 0.0)

    validb = val > 0.5
    scale = jnp.float32(1.0 / math.sqrt(head))
    agg_parts = []
    aw_parts = []
    for h in range(H):
        hs = slice(h * head, (h + 1) * head)
        qph = qp[:, h * 2 * head:(h + 1) * 2 * head]
        s2 = jnp.sum(hrel * qph[:, None, :], axis=2)
        s1 = jnp.sum(ktg[:, :, hs] * q[:, None, hs], axis=2)
        s = (s1 + s2) * scale
        s = jnp.where(validb, s, -1e9)
        mx = jnp.max(s, axis=1, keepdims=True)
        ex = jnp.exp(s - mx)
        attn = ex / jnp.sum(ex, axis=1, keepdims=True)
        agg_parts.append(jnp.sum(vtg[:, :, hs] * attn[:, :, None], axis=1))
        aw_parts.append(jnp.sum(hrel * attn[:, :, None], axis=1))
    agg1 = jnp.concatenate(agg_parts, axis=1)
    aw = jnp.concatenate(aw_parts, axis=1)
    agg = agg1 + jnp.dot(aw, wv2bd[...], preferred_element_type=jnp.float32)
    outp = jnp.dot(agg, wout_t[...], preferred_element_type=jnp.float32) + bout[...]
    has_nb = jnp.max(val, axis=1, keepdims=True) > 0.5
    enh = jnp.where(has_nb, bf + outp, bf)
    mu = jnp.mean(enh, axis=1, keepdims=True)
    var = jnp.mean((enh - mu) ** 2, axis=1, keepdims=True)
    out_r[0] = gamma[...] * (enh - mu) / jnp.sqrt(var + 1e-5) + beta[...]


def _make_sc_gather(rows, width, nw):
    per_w = rows // nw
    nch = per_w // CH
    mesh = plsc.VectorSubcoreMesh(core_axis_name="c", subcore_axis_name="s")

    @functools.partial(
        pl.kernel, mesh=mesh,
        out_type=jax.ShapeDtypeStruct((rows, width), jnp.float32),
        scratch_types=[pltpu.VMEM((nch, CH), jnp.int32),
                       pltpu.VMEM((CH, width), jnp.float32),
                       pltpu.SemaphoreType.DMA],
    )
    def sc_gather(idx2_hbm, tab_hbm, out_hbm, idx_v, buf, s1):
        nc = 2
        wid = lax.axis_index("s") * nc + lax.axis_index("c")
        base = wid * per_w
        pltpu.sync_copy(idx2_hbm.at[pl.ds(wid * nch, nch)], idx_v)

        def body(j, carry):
            off = base + j * CH
            pltpu.async_copy(tab_hbm.at[idx_v.at[j]], buf, s1).wait()
            pltpu.sync_copy(buf, out_hbm.at[pl.ds(off, CH)])
            return carry

        lax.fori_loop(0, nch, body, 0)

    return sc_gather


def kernel(features, coords, W_fp, b_fp, W_pe1, b_pe1, W_pe2, b_pe2,
           W_in, b_in, W_out, b_out, gamma, beta):
    B, N, IN_DIM = features.shape
    E = W_fp.shape[0]
    head = E // H
    f32 = jnp.float32

    Wq = W_in[:E]
    Wk = W_in[E:2 * E]
    Wv = W_in[2 * E:]
    bq = b_in[:E]
    bk = b_in[E:2 * E]
    bv = b_in[2 * E:]
    Wk2 = Wk @ W_pe2
    Wv2 = Wv @ W_pe2
    ktb = bk + b_pe2 @ Wk.T
    vtb = bv + b_pe2 @ Wv.T
    Wqp = jnp.zeros((E, 2 * E), f32)
    Wv2bd = jnp.zeros((2 * E, E), f32)
    for h in range(H):
        hs = slice(h * head, (h + 1) * head)
        ps = slice(h * 2 * head, (h + 1) * 2 * head)
        Wqp = Wqp.at[hs, ps].set(Wk2[hs, :])
        Wv2bd = Wv2bd.at[ps, hs].set(Wv2[hs, :].T)

    cds8 = jnp.concatenate([coords, jnp.zeros((B, N, 5), f32)], axis=-1)
    crow = jnp.swapaxes(cds8, 1, 2)

    nb_a = N // BM_A
    row2 = lambda b, i: (b, i, 0)
    w2 = lambda b, i: (0, 0)
    outs_a = pl.pallas_call(
        _kernel_a,
        grid=(B, nb_a),
        in_specs=[
            pl.BlockSpec((1, BM_A, IN_DIM), row2),
            pl.BlockSpec((1, 8, N), lambda b, i: (b, 0, 0)),
            pl.BlockSpec((1, BM_A, 8), row2),
            pl.BlockSpec((IN_DIM, E), w2),
            pl.BlockSpec((E, E), w2),
            pl.BlockSpec((E, E), w2),
            pl.BlockSpec((E, E), w2),
            pl.BlockSpec((E, 2 * E), w2),
            pl.BlockSpec((1, E), w2),
            pl.BlockSpec((1, E), w2),
            pl.BlockSpec((1, E), w2),
            pl.BlockSpec((1, E), w2),
        ],
        out_specs=[
            pl.BlockSpec((1, BM_A, E), row2),
            pl.BlockSpec((1, BM_A, E), row2),
            pl.BlockSpec((1, BM_A, E), row2),
            pl.BlockSpec((1, BM_A, E), row2),
            pl.BlockSpec((1, BM_A, 2 * E), row2),
            pl.BlockSpec((1, BM_A, K), row2),
            pl.BlockSpec((1, BM_A, K), row2),
            pl.BlockSpec((1, BM_A, K), row2),
        ],
        out_shape=[
            jax.ShapeDtypeStruct((B, N, E), f32),
            jax.ShapeDtypeStruct((B, N, E), f32),
            jax.ShapeDtypeStruct((B, N, E), f32),
            jax.ShapeDtypeStruct((B, N, E), f32),
            jax.ShapeDtypeStruct((B, N, 2 * E), f32),
            jax.ShapeDtypeStruct((B, N, K), jnp.int32),
            jax.ShapeDtypeStruct((B, N, K), f32),
            jax.ShapeDtypeStruct((B, N, K), f32),
        ],
    )(features, crow, cds8,
      W_fp.T, Wq.T, Wk.T, Wv.T, Wqp,
      b_fp[None, :], bq[None, :], ktb[None, :], vtb[None, :])
    bf, q, kt, vt, qp, idxg, knn, valf = outs_a

    rows_b = N * K
    width = 2 * E + 128
    comb = jnp.concatenate(
        [kt, vt, coords, jnp.zeros((B, N, 125), f32)], axis=-1)
    sc_gather = _make_sc_gather(rows_b, width, 32)
    nb_c = N // BM_C
    wc = lambda i: (0, 0)
    kernel_c_call = pl.pallas_call(
        _kernel_c,
        grid=(nb_c,),
        in_specs=[
            pl.BlockSpec((1, BM_C, E), lambda i: (0, i, 0)),
            pl.BlockSpec((1, BM_C, 2 * E), lambda i: (0, i, 0)),
            pl.BlockSpec((1, BM_C, E), lambda i: (0, i, 0)),
            pl.BlockSpec((1, BM_C, K, 2 * E + 128), lambda i: (0, i, 0, 0)),
            pl.BlockSpec((1, BM_C, 8), lambda i: (0, i, 0)),
            pl.BlockSpec((1, BM_C, K), lambda i: (0, i, 0)),
            pl.BlockSpec((1, BM_C, K), lambda i: (0, i, 0)),
            pl.BlockSpec((4, 2 * head), wc),
            pl.BlockSpec((1, 2 * head), wc),
            pl.BlockSpec((2 * E, E), wc),
            pl.BlockSpec((E, E), wc),
            pl.BlockSpec((1, E), wc),
            pl.BlockSpec((1, E), wc),
            pl.BlockSpec((1, E), wc),
        ],
        out_specs=pl.BlockSpec((1, BM_C, E), lambda i: (0, i, 0)),
        out_shape=jax.ShapeDtypeStruct((1, N, E), f32),
    )
    gathered = []
    for bb in range(B):
        gathered.append(sc_gather(
            idxg[bb].reshape(rows_b // CH, CH), comb[bb]))
    outs = []
    for bb in range(B):
        out_b = kernel_c_call(
            q[bb:bb + 1], qp[bb:bb + 1], bf[bb:bb + 1],
            gathered[bb].reshape(1, N, K, width), cds8[bb:bb + 1],
            knn[bb:bb + 1], valf[bb:bb + 1],
            W_pe1.T, b_pe1[None, :], Wv2bd, W_out.T, b_out[None, :],
            gamma[None, :], beta[None, :])
        outs.append(out_b)
    return jnp.concatenate(outs, axis=0)


# speedup vs baseline: 1.1239x; 1.1239x over previous
import functools
import math

import jax
import jax.numpy as jnp
from jax import lax
from jax.experimental import pallas as pl
from jax.experimental.pallas import tpu as pltpu
from jax.experimental.pallas import tpu_sc as plsc

K = 16
H = 4
RADIUS = 50.0

BM_A = 256
BM_C = 256
CH = 128


def _kernel_a(feats, crow, ccol, wfp_t, wq_t, wkt_t, wvt_t, wqp,
              bfp, bq, bkt, bvt,
              bf_o, q_o, qp_o, comb_o, idx_o, knn_o, val_o):
    i = pl.program_id(0)
    n_total = crow.shape[2]

    x = feats[0]
    bf = jnp.dot(x, wfp_t[...], preferred_element_type=jnp.float32) + bfp[...]
    q = jnp.dot(bf, wq_t[...], preferred_element_type=jnp.float32) + bq[...]
    kt = jnp.dot(bf, wkt_t[...], preferred_element_type=jnp.float32) + bkt[...]
    vt = jnp.dot(bf, wvt_t[...], preferred_element_type=jnp.float32) + bvt[...]
    qp = jnp.dot(q, wqp[...], preferred_element_type=jnp.float32)
    bf_o[0] = bf
    q_o[0] = q
    qp_o[0] = qp

    cr = crow[0]
    cc = ccol[0]
    comb_o[0] = jnp.concatenate(
        [kt, vt, cc, jnp.zeros((cc.shape[0], 120), jnp.float32)], axis=1)
    xr = cr[0:1, :]
    yr = cr[1:2, :]
    zr = cr[2:3, :]
    xc = cc[:, 0:1]
    yc = cc[:, 1:2]
    zc = cc[:, 2:3]
    dx = xc - xr
    dy = yc - yr
    dz = zc - zr
    sq = dx * dx + dy * dy
    pos = sq > 0.0
    spatial = jnp.where(pos, jnp.sqrt(jnp.where(pos, sq, 1.0)), 0.0)
    dist = spatial + 0.3 * jnp.abs(dz)

    jj = lax.broadcasted_iota(jnp.int32, dist.shape, 1)
    nn = lax.broadcasted_iota(jnp.int32, dist.shape, 0) + i * BM_A
    inf = jnp.float32(jnp.inf)
    masked = (jj == nn) | (zr > zc) | (spatial > RADIUS)
    d = jnp.where(masked, inf, dist)

    idx_cols = []
    knn_cols = []
    val_cols = []
    for _ in range(K):
        m = jnp.min(d, axis=1, keepdims=True)
        eq = d == m
        cand = jnp.where(eq, jj, n_total)
        sel = jnp.min(cand, axis=1, keepdims=True)
        valid = m < inf
        idx_cols.append(sel)
        knn_cols.append(jnp.where(valid, m, 0.0))
        val_cols.append(valid.astype(jnp.float32))
        d = jnp.where(jj == sel, inf, d)
    idx_o[0] = jnp.concatenate(idx_cols, axis=1)
    knn_o[0] = jnp.concatenate(knn_cols, axis=1)
    val_o[0] = jnp.concatenate(val_cols, axis=1)


def _kernel_c(q_r, qp_r, bf_r, g_r, own_r,
              knn_r, val_r,
              wpe1_t, bpe1, wv2bd, wout_t, bout, gamma, beta, out_r):
    e = q_r.shape[2]
    head = e // H
    q = q_r[0]
    qp = qp_r[0]
    bf = bf_r[0]
    g = g_r[0]
    ktg = g[:, :, 0:e]
    vtg = g[:, :, e:2 * e]
    own = own_r[0]
    knn = knn_r[0]
    val = val_r[0]

    relx = g[:, :, 2 * e:2 * e + 1] - own[:, None, 0:1]
    rely = g[:, :, 2 * e + 1:2 * e + 2] - own[:, None, 1:2]
    relz = g[:, :, 2 * e + 2:2 * e + 3] - own[:, None, 2:3]
    acc = jnp.broadcast_to(jnp.reshape(bpe1[...], (1, 1, 2 * head)),
                           (q.shape[0], K, 2 * head))
    feats4 = [relx, rely, relz, knn[:, :, None]]
    for f in range(4):
        acc = acc + feats4[f] * jnp.reshape(wpe1_t[f:f + 1, :], (1, 1, 2 * head))
    hrel = jnp.maximum(acc, 0.0)

    validb = val > 0.5
    scale = jnp.float32(1.0 / math.sqrt(head))
    agg_parts = []
    aw_parts = []
    for h in range(H):
        hs = slice(h * head, (h + 1) * head)
        qph = qp[:, h * 2 * head:(h + 1) * 2 * head]
        s2 = jnp.sum(hrel * qph[:, None, :], axis=2)
        s1 = jnp.sum(ktg[:, :, hs] * q[:, None, hs], axis=2)
        s = (s1 + s2) * scale
        s = jnp.where(validb, s, -1e9)
        mx = jnp.max(s, axis=1, keepdims=True)
        ex = jnp.exp(s - mx)
        attn = ex / jnp.sum(ex, axis=1, keepdims=True)
        agg_parts.append(jnp.sum(vtg[:, :, hs] * attn[:, :, None], axis=1))
        aw_parts.append(jnp.sum(hrel * attn[:, :, None], axis=1))
    agg1 = jnp.concatenate(agg_parts, axis=1)
    aw = jnp.concatenate(aw_parts, axis=1)
    agg = agg1 + jnp.dot(aw, wv2bd[...], preferred_element_type=jnp.float32)
    outp = jnp.dot(agg, wout_t[...], preferred_element_type=jnp.float32) + bout[...]
    has_nb = jnp.max(val, axis=1, keepdims=True) > 0.5
    enh = jnp.where(has_nb, bf + outp, bf)
    mu = jnp.mean(enh, axis=1, keepdims=True)
    var = jnp.mean((enh - mu) ** 2, axis=1, keepdims=True)
    out_r[0] = gamma[...] * (enh - mu) / jnp.sqrt(var + 1e-5) + beta[...]


def _make_sc_gather(rows, width, nw):
    per_w = rows // nw
    nch = per_w // CH
    mesh = plsc.VectorSubcoreMesh(core_axis_name="c", subcore_axis_name="s")

    @functools.partial(
        pl.kernel, mesh=mesh,
        out_type=jax.ShapeDtypeStruct((rows, width), jnp.float32),
        scratch_types=[pltpu.VMEM((nch, CH), jnp.int32),
                       pltpu.VMEM((CH, width), jnp.float32),
                       pltpu.SemaphoreType.DMA],
    )
    def sc_gather(idx2_hbm, tab_hbm, out_hbm, idx_v, buf, s1):
        nc = 2
        wid = lax.axis_index("s") * nc + lax.axis_index("c")
        base = wid * per_w
        pltpu.sync_copy(idx2_hbm.at[pl.ds(wid * nch, nch)], idx_v)

        def body(j, carry):
            off = base + j * CH
            pltpu.async_copy(tab_hbm.at[idx_v.at[j]], buf, s1).wait()
            pltpu.sync_copy(buf, out_hbm.at[pl.ds(off, CH)])
            return carry

        lax.fori_loop(0, nch, body, 0)

    return sc_gather


def kernel(features, coords, W_fp, b_fp, W_pe1, b_pe1, W_pe2, b_pe2,
           W_in, b_in, W_out, b_out, gamma, beta):
    B, N, IN_DIM = features.shape
    E = W_fp.shape[0]
    head = E // H
    f32 = jnp.float32

    Wq = W_in[:E]
    Wk = W_in[E:2 * E]
    Wv = W_in[2 * E:]
    bq = b_in[:E]
    bk = b_in[E:2 * E]
    bv = b_in[2 * E:]
    Wk2 = Wk @ W_pe2
    Wv2 = Wv @ W_pe2
    ktb = bk + b_pe2 @ Wk.T
    vtb = bv + b_pe2 @ Wv.T
    Wqp = jnp.zeros((E, 2 * E), f32)
    Wv2bd = jnp.zeros((2 * E, E), f32)
    for h in range(H):
        hs = slice(h * head, (h + 1) * head)
        ps = slice(h * 2 * head, (h + 1) * 2 * head)
        Wqp = Wqp.at[hs, ps].set(Wk2[hs, :])
        Wv2bd = Wv2bd.at[ps, hs].set(Wv2[hs, :].T)

    cds8 = jnp.concatenate([coords, jnp.zeros((B, N, 5), f32)], axis=-1)
    crow = jnp.swapaxes(cds8, 1, 2)

    nb_a = N // BM_A
    width = 2 * E + 128
    rowa = lambda i: (0, i, 0)
    wa = lambda i: (0, 0)
    a_call = pl.pallas_call(
        _kernel_a,
        grid=(nb_a,),
        in_specs=[
            pl.BlockSpec((1, BM_A, IN_DIM), rowa),
            pl.BlockSpec((1, 8, N), lambda i: (0, 0, 0)),
            pl.BlockSpec((1, BM_A, 8), rowa),
            pl.BlockSpec((IN_DIM, E), wa),
            pl.BlockSpec((E, E), wa),
            pl.BlockSpec((E, E), wa),
            pl.BlockSpec((E, E), wa),
            pl.BlockSpec((E, 2 * E), wa),
            pl.BlockSpec((1, E), wa),
            pl.BlockSpec((1, E), wa),
            pl.BlockSpec((1, E), wa),
            pl.BlockSpec((1, E), wa),
        ],
        out_specs=[
            pl.BlockSpec((1, BM_A, E), rowa),
            pl.BlockSpec((1, BM_A, E), rowa),
            pl.BlockSpec((1, BM_A, 2 * E), rowa),
            pl.BlockSpec((1, BM_A, width), rowa),
            pl.BlockSpec((1, BM_A, K), rowa),
            pl.BlockSpec((1, BM_A, K), rowa),
            pl.BlockSpec((1, BM_A, K), rowa),
        ],
        out_shape=[
            jax.ShapeDtypeStruct((1, N, E), f32),
            jax.ShapeDtypeStruct((1, N, E), f32),
            jax.ShapeDtypeStruct((1, N, 2 * E), f32),
            jax.ShapeDtypeStruct((1, N, width), f32),
            jax.ShapeDtypeStruct((1, N, K), jnp.int32),
            jax.ShapeDtypeStruct((1, N, K), f32),
            jax.ShapeDtypeStruct((1, N, K), f32),
        ],
    )
    wfp_t = W_fp.T
    wq_t = Wq.T
    wkt_t = Wk.T
    wvt_t = Wv.T
    bfp2 = b_fp[None, :]
    bq2 = bq[None, :]
    ktb2 = ktb[None, :]
    vtb2 = vtb[None, :]

    rows_b = N * K
    sc_gather = _make_sc_gather(rows_b, width, 32)
    nb_c = N // BM_C
    wc = lambda i: (0, 0)
    kernel_c_call = pl.pallas_call(
        _kernel_c,
        grid=(nb_c,),
        in_specs=[
            pl.BlockSpec((1, BM_C, E), lambda i: (0, i, 0)),
            pl.BlockSpec((1, BM_C, 2 * E), lambda i: (0, i, 0)),
            pl.BlockSpec((1, BM_C, E), lambda i: (0, i, 0)),
            pl.BlockSpec((1, BM_C, K, 2 * E + 128), lambda i: (0, i, 0, 0)),
            pl.BlockSpec((1, BM_C, 8), lambda i: (0, i, 0)),
            pl.BlockSpec((1, BM_C, K), lambda i: (0, i, 0)),
            pl.BlockSpec((1, BM_C, K), lambda i: (0, i, 0)),
            pl.BlockSpec((4, 2 * head), wc),
            pl.BlockSpec((1, 2 * head), wc),
            pl.BlockSpec((2 * E, E), wc),
            pl.BlockSpec((E, E), wc),
            pl.BlockSpec((1, E), wc),
            pl.BlockSpec((1, E), wc),
            pl.BlockSpec((1, E), wc),
        ],
        out_specs=pl.BlockSpec((1, BM_C, E), lambda i: (0, i, 0)),
        out_shape=jax.ShapeDtypeStruct((1, N, E), f32),
    )
    per_batch = []
    for bb in range(B):
        bf_b, q_b, qp_b, comb_b, idx_b, knn_b, val_b = a_call(
            features[bb:bb + 1], crow[bb:bb + 1], cds8[bb:bb + 1],
            wfp_t, wq_t, wkt_t, wvt_t, Wqp, bfp2, bq2, ktb2, vtb2)
        g_b = sc_gather(idx_b.reshape(rows_b // CH, CH), comb_b[0])
        per_batch.append((bf_b, q_b, qp_b, g_b, knn_b, val_b))
    outs = []
    for bb in range(B):
        bf_b, q_b, qp_b, g_b, knn_b, val_b = per_batch[bb]
        out_b = kernel_c_call(
            q_b, qp_b, bf_b,
            g_b.reshape(1, N, K, width), cds8[bb:bb + 1],
            knn_b, val_b,
            W_pe1.T, b_pe1[None, :], Wv2bd, W_out.T, b_out[None, :],
            gamma[None, :], beta[None, :])
        outs.append(out_b)
    return jnp.concatenate(outs, axis=0)
